# Initial kernel scaffold; baseline (speedup 1.0000x reference)
import jax
import jax.numpy as jnp
from jax.experimental import pallas as pl


def kernel(x, prototypes):
    raise NotImplementedError("write your pallas kernel here")


# speedup vs baseline: 9.9646x; 9.9646x over previous
import jax
import jax.numpy as jnp
from jax import lax
from jax.experimental import pallas as pl
from jax.experimental.pallas import tpu as pltpu
from jax.experimental.pallas import tpu_sc as plsc

N_TOKENS = 16384
NPROTO = 8192
DIM = 32
COMMITMENT_COST = 0.25

BLK = 128
NBLK = N_TOKENS // BLK

NUM_WORKERS = 32
BPW = N_TOKENS // NUM_WORKERS
IDX_CHUNK = 128
NCHUNK = BPW // IDX_CHUNK


def _bf16_rne(v):
    u = lax.bitcast_convert_type(v, jnp.int32)
    lsb = lax.shift_right_logical(u, 16) & 1
    r = (u + 0x7FFF + lsb) & jnp.int32(-65536)
    return lax.bitcast_convert_type(r, jnp.float32)


def _argmin_body(x_ref, p_ref, sx_ref, sp_ref, idx_ref, loss_ref):
    i = pl.program_id(0)
    xb = x_ref[...]
    p = p_ref[...]
    m = lax.dot_general(xb, p, (((1,), (1,)), ((), ())),
                        preferred_element_type=jnp.float32)
    d = (sx_ref[...] + sp_ref[...]) - 2.0 * m

    HALF = NPROTO // 2
    vals, idxs = [], []
    for s in range(2):
        ds = d[:, s * HALF:(s + 1) * HALF]
        vmin = jnp.min(ds, axis=1, keepdims=True)
        iota = lax.broadcasted_iota(jnp.int32, ds.shape, 1) + jnp.int32(s * HALF)
        imin = jnp.min(jnp.where(ds == vmin, iota, jnp.int32(NPROTO)),
                       axis=1, keepdims=True)
        vals.append(vmin)
        idxs.append(imin)
    tb = vals[1] < _bf16_rne(vals[0])
    dmin = jnp.where(tb, vals[1], vals[0])
    idx = jnp.where(tb, idxs[1], idxs[0])
    idx_ref[...] = idx

    @pl.when(i == 0)
    def _init():
        loss_ref[...] = jnp.zeros((1, 1), jnp.float32)

    loss_ref[...] += jnp.sum(dmin).reshape(1, 1)

    @pl.when(i == NBLK - 1)
    def _finish():
        mean_sq = loss_ref[...] / (N_TOKENS * DIM)
        loss_ref[...] = mean_sq + COMMITMENT_COST * mean_sq


def _argmin_call(x, prototypes, sx, sp):
    return pl.pallas_call(
        _argmin_body,
        grid=(NBLK,),
        in_specs=[
            pl.BlockSpec((BLK, DIM), lambda i: (i, 0)),
            pl.BlockSpec((NPROTO, DIM), lambda i: (0, 0)),
            pl.BlockSpec((BLK, 1), lambda i: (i, 0)),
            pl.BlockSpec((1, NPROTO), lambda i: (0, 0)),
        ],
        out_specs=[
            pl.BlockSpec((BLK, 1), lambda i: (i, 0)),
            pl.BlockSpec((1, 1), lambda i: (0, 0)),
        ],
        out_shape=[
            jax.ShapeDtypeStruct((N_TOKENS, 1), jnp.int32),
            jax.ShapeDtypeStruct((1, 1), jnp.float32),
        ],
    )(x, prototypes, sx, sp)


def _sc_gather_body(table_hbm, idx_hbm, out_hbm, idx_v, rows_v, sem):
    wid = lax.axis_index("s") * 2 + lax.axis_index("c")
    base = wid * BPW
    pltpu.sync_copy(idx_hbm.at[wid], idx_v)
    copies = [
        pltpu.async_copy(table_hbm.at[idx_v.at[j]],
                         rows_v.at[pl.ds(j * IDX_CHUNK, IDX_CHUNK)], sem)
        for j in range(NCHUNK)
    ]
    for c in copies:
        c.wait()
    pltpu.sync_copy(rows_v, out_hbm.at[pl.ds(base, BPW)])


def _gather_call(prototypes, idx):
    mesh = plsc.VectorSubcoreMesh(core_axis_name="c", subcore_axis_name="s")
    f = pl.kernel(
        _sc_gather_body,
        out_type=jax.ShapeDtypeStruct((N_TOKENS, DIM), jnp.float32),
        mesh=mesh,
        scratch_types=[
            pltpu.VMEM((NCHUNK, IDX_CHUNK), jnp.int32),
            pltpu.VMEM((BPW, DIM), jnp.float32),
            pltpu.SemaphoreType.DMA,
        ],
        compiler_params=pltpu.CompilerParams(use_tc_tiling_on_sc=False),
    )
    return f(prototypes, idx)


def kernel(x, prototypes):
    sx = jnp.sum(x ** 2, axis=1, keepdims=True)
    sp = jnp.sum(prototypes ** 2, axis=1)
    idx2, loss = _argmin_call(x, prototypes, sx, sp.reshape(1, NPROTO))
    idx = idx2.reshape(NUM_WORKERS, NCHUNK, IDX_CHUNK)
    quantized_st = _gather_call(prototypes, idx)
    return (quantized_st, loss[0, 0], prototypes)


# speedup vs baseline: 10.6951x; 1.0733x over previous
import jax
import jax.numpy as jnp
from jax import lax
from jax.experimental import pallas as pl
from jax.experimental.pallas import tpu as pltpu
from jax.experimental.pallas import tpu_sc as plsc

N_TOKENS = 16384
NPROTO = 8192
DIM = 32
COMMITMENT_COST = 0.25

BLK = 256
NBLK = N_TOKENS // BLK

NUM_WORKERS = 32
BPW = N_TOKENS // NUM_WORKERS
IDX_CHUNK = 128
NCHUNK = BPW // IDX_CHUNK


def _bf16_rne(v):
    u = lax.bitcast_convert_type(v, jnp.int32)
    lsb = lax.shift_right_logical(u, 16) & 1
    r = (u + 0x7FFF + lsb) & jnp.int32(-65536)
    return lax.bitcast_convert_type(r, jnp.float32)


def _argmin_body(x_ref, p2_ref, sx_ref, sp_ref, idx_ref, loss_ref):
    i = pl.program_id(0)
    xb = x_ref[...]
    p2 = p2_ref[...]
    m2 = lax.dot_general(xb, p2, (((1,), (1,)), ((), ())),
                         preferred_element_type=jnp.float32)
    d = (sx_ref[...] + sp_ref[...]) - m2

    HALF = NPROTO // 2
    vals, idxs = [], []
    for s in range(2):
        ds = d[:, s * HALF:(s + 1) * HALF]
        vmin = jnp.min(ds, axis=1, keepdims=True)
        iota = lax.broadcasted_iota(jnp.int32, ds.shape, 1) + jnp.int32(s * HALF)
        imin = jnp.min(jnp.where(ds == vmin, iota, jnp.int32(NPROTO)),
                       axis=1, keepdims=True)
        vals.append(vmin)
        idxs.append(imin)
    tb = vals[1] < _bf16_rne(vals[0])
    dmin = jnp.where(tb, vals[1], vals[0])
    idx = jnp.where(tb, idxs[1], idxs[0])
    idx_ref[...] = idx

    @pl.when(i == 0)
    def _init():
        loss_ref[...] = jnp.zeros((1, 1), jnp.float32)

    loss_ref[...] += jnp.sum(dmin).reshape(1, 1)

    @pl.when(i == NBLK - 1)
    def _finish():
        mean_sq = loss_ref[...] / (N_TOKENS * DIM)
        loss_ref[...] = mean_sq + COMMITMENT_COST * mean_sq


def _argmin_call(x, prototypes, sx, sp):
    return pl.pallas_call(
        _argmin_body,
        grid=(NBLK,),
        in_specs=[
            pl.BlockSpec((BLK, DIM), lambda i: (i, 0)),
            pl.BlockSpec((NPROTO, DIM), lambda i: (0, 0)),
            pl.BlockSpec((BLK, 1), lambda i: (i, 0)),
            pl.BlockSpec((1, NPROTO), lambda i: (0, 0)),
        ],
        out_specs=[
            pl.BlockSpec((BLK, 1), lambda i: (i, 0)),
            pl.BlockSpec((1, 1), lambda i: (0, 0)),
        ],
        out_shape=[
            jax.ShapeDtypeStruct((N_TOKENS, 1), jnp.int32),
            jax.ShapeDtypeStruct((1, 1), jnp.float32),
        ],
    )(x, prototypes, sx, sp)


def _sc_gather_body(table_hbm, idx_hbm, out_hbm, idx_v, rows_v, sem):
    wid = lax.axis_index("s") * 2 + lax.axis_index("c")
    base = wid * BPW
    pltpu.sync_copy(idx_hbm.at[wid], idx_v)
    copies = [
        pltpu.async_copy(table_hbm.at[idx_v.at[j]],
                         rows_v.at[pl.ds(j * IDX_CHUNK, IDX_CHUNK)], sem)
        for j in range(NCHUNK)
    ]
    for c in copies:
        c.wait()
    pltpu.sync_copy(rows_v, out_hbm.at[pl.ds(base, BPW)])


def _gather_call(prototypes, idx):
    mesh = plsc.VectorSubcoreMesh(core_axis_name="c", subcore_axis_name="s")
    f = pl.kernel(
        _sc_gather_body,
        out_type=jax.ShapeDtypeStruct((N_TOKENS, DIM), jnp.float32),
        mesh=mesh,
        scratch_types=[
            pltpu.VMEM((NCHUNK, IDX_CHUNK), jnp.int32),
            pltpu.VMEM((BPW, DIM), jnp.float32),
            pltpu.SemaphoreType.DMA,
        ],
        compiler_params=pltpu.CompilerParams(use_tc_tiling_on_sc=False),
    )
    return f(prototypes, idx)


def kernel(x, prototypes):
    sx = jnp.sum(x ** 2, axis=1, keepdims=True)
    sp = jnp.sum(prototypes ** 2, axis=1)
    idx2, loss = _argmin_call(x, prototypes + prototypes, sx,
                              sp.reshape(1, NPROTO))
    idx = idx2.reshape(NUM_WORKERS, NCHUNK, IDX_CHUNK)
    quantized_st = _gather_call(prototypes, idx)
    return (quantized_st, loss[0, 0], prototypes)


# speedup vs baseline: 11.1281x; 1.0405x over previous
import jax
import jax.numpy as jnp
from jax import lax
from jax.experimental import pallas as pl
from jax.experimental.pallas import tpu as pltpu
from jax.experimental.pallas import tpu_sc as plsc

N_TOKENS = 16384
NPROTO = 8192
DIM = 32
COMMITMENT_COST = 0.25

BLK = 512
NBLK = N_TOKENS // BLK

NUM_WORKERS = 32
BPW = N_TOKENS // NUM_WORKERS
IDX_CHUNK = 128
NCHUNK = BPW // IDX_CHUNK


def _bf16_rne(v):
    u = lax.bitcast_convert_type(v, jnp.int32)
    lsb = lax.shift_right_logical(u, 16) & 1
    r = (u + 0x7FFF + lsb) & jnp.int32(-65536)
    return lax.bitcast_convert_type(r, jnp.float32)


def _argmin_body(x_ref, p2_ref, sx_ref, sp_ref, idx_ref, loss_ref):
    i = pl.program_id(0)
    xb = x_ref[...]
    p2 = p2_ref[...]
    m2 = lax.dot_general(xb, p2, (((1,), (1,)), ((), ())),
                         preferred_element_type=jnp.float32)
    d = (sx_ref[...] + sp_ref[...]) - m2

    HALF = NPROTO // 2
    vals, idxs = [], []
    for s in range(2):
        ds = d[:, s * HALF:(s + 1) * HALF]
        vmin = jnp.min(ds, axis=1, keepdims=True)
        iota = lax.broadcasted_iota(jnp.int32, ds.shape, 1) + jnp.int32(s * HALF)
        imin = jnp.min(jnp.where(ds == vmin, iota, jnp.int32(NPROTO)),
                       axis=1, keepdims=True)
        vals.append(vmin)
        idxs.append(imin)
    tb = vals[1] < _bf16_rne(vals[0])
    dmin = jnp.where(tb, vals[1], vals[0])
    idx = jnp.where(tb, idxs[1], idxs[0])
    idx_ref[...] = idx

    @pl.when(i == 0)
    def _init():
        loss_ref[...] = jnp.zeros((1, 1), jnp.float32)

    loss_ref[...] += jnp.sum(dmin).reshape(1, 1)

    @pl.when(i == NBLK - 1)
    def _finish():
        mean_sq = loss_ref[...] / (N_TOKENS * DIM)
        loss_ref[...] = mean_sq + COMMITMENT_COST * mean_sq


def _argmin_call(x, prototypes, sx, sp):
    return pl.pallas_call(
        _argmin_body,
        grid=(NBLK,),
        in_specs=[
            pl.BlockSpec((BLK, DIM), lambda i: (i, 0)),
            pl.BlockSpec((NPROTO, DIM), lambda i: (0, 0)),
            pl.BlockSpec((BLK, 1), lambda i: (i, 0)),
            pl.BlockSpec((1, NPROTO), lambda i: (0, 0)),
        ],
        out_specs=[
            pl.BlockSpec((BLK, 1), lambda i: (i, 0)),
            pl.BlockSpec((1, 1), lambda i: (0, 0)),
        ],
        out_shape=[
            jax.ShapeDtypeStruct((N_TOKENS, 1), jnp.int32),
            jax.ShapeDtypeStruct((1, 1), jnp.float32),
        ],
    )(x, prototypes, sx, sp)


def _sc_gather_body(table_hbm, idx_hbm, out_hbm, idx_v, rows_v, sem):
    wid = lax.axis_index("s") * 2 + lax.axis_index("c")
    base = wid * BPW
    pltpu.sync_copy(idx_hbm.at[wid], idx_v)
    copies = [
        pltpu.async_copy(table_hbm.at[idx_v.at[j]],
                         rows_v.at[pl.ds(j * IDX_CHUNK, IDX_CHUNK)], sem)
        for j in range(NCHUNK)
    ]
    for c in copies:
        c.wait()
    pltpu.sync_copy(rows_v, out_hbm.at[pl.ds(base, BPW)])


def _gather_call(prototypes, idx):
    mesh = plsc.VectorSubcoreMesh(core_axis_name="c", subcore_axis_name="s")
    f = pl.kernel(
        _sc_gather_body,
        out_type=jax.ShapeDtypeStruct((N_TOKENS, DIM), jnp.float32),
        mesh=mesh,
        scratch_types=[
            pltpu.VMEM((NCHUNK, IDX_CHUNK), jnp.int32),
            pltpu.VMEM((BPW, DIM), jnp.float32),
            pltpu.SemaphoreType.DMA,
        ],
        compiler_params=pltpu.CompilerParams(use_tc_tiling_on_sc=False),
    )
    return f(prototypes, idx)


def kernel(x, prototypes):
    sx = jnp.sum(x ** 2, axis=1, keepdims=True)
    sp = jnp.sum(prototypes ** 2, axis=1)
    idx2, loss = _argmin_call(x, prototypes + prototypes, sx,
                              sp.reshape(1, NPROTO))
    idx = idx2.reshape(NUM_WORKERS, NCHUNK, IDX_CHUNK)
    quantized_st = _gather_call(prototypes, idx)
    return (quantized_st, loss[0, 0], prototypes)


# speedup vs baseline: 11.5590x; 1.0387x over previous
import jax
import jax.numpy as jnp
from jax import lax
from jax.experimental import pallas as pl
from jax.experimental.pallas import tpu as pltpu
from jax.experimental.pallas import tpu_sc as plsc

N_TOKENS = 16384
NPROTO = 8192
DIM = 32
COMMITMENT_COST = 0.25

BLK = 512
NBLK = N_TOKENS // BLK

NUM_WORKERS = 32
BPW = N_TOKENS // NUM_WORKERS
IDX_CHUNK = 128
NCHUNK = BPW // IDX_CHUNK


def _bf16_rne(v):
    u = lax.bitcast_convert_type(v, jnp.int32)
    lsb = lax.shift_right_logical(u, 16) & 1
    r = (u + 0x7FFF + lsb) & jnp.int32(-65536)
    return lax.bitcast_convert_type(r, jnp.float32)


def _argmin_body(x_ref, p2_ref, sx_ref, sp_ref, idx_ref, loss_ref):
    i = pl.program_id(0)
    xb = x_ref[...]
    p2t = p2_ref[...]
    m2 = lax.dot_general(xb, p2t, (((1,), (0,)), ((), ())),
                         preferred_element_type=jnp.float32)
    d = (sx_ref[...] + sp_ref[...]) - m2

    HALF = NPROTO // 2
    vals, idxs = [], []
    for s in range(2):
        ds = d[:, s * HALF:(s + 1) * HALF]
        vmin = jnp.min(ds, axis=1, keepdims=True)
        iota = lax.broadcasted_iota(jnp.int32, ds.shape, 1) + jnp.int32(s * HALF)
        imin = jnp.min(jnp.where(ds == vmin, iota, jnp.int32(NPROTO)),
                       axis=1, keepdims=True)
        vals.append(vmin)
        idxs.append(imin)
    tb = vals[1] < _bf16_rne(vals[0])
    dmin = jnp.where(tb, vals[1], vals[0])
    idx = jnp.where(tb, idxs[1], idxs[0])
    idx_ref[...] = idx

    @pl.when(i == 0)
    def _init():
        loss_ref[...] = jnp.zeros((1, 1), jnp.float32)

    loss_ref[...] += jnp.sum(dmin).reshape(1, 1)

    @pl.when(i == NBLK - 1)
    def _finish():
        mean_sq = loss_ref[...] / (N_TOKENS * DIM)
        loss_ref[...] = mean_sq + COMMITMENT_COST * mean_sq


def _argmin_call(x, prototypes, sx, sp):
    return pl.pallas_call(
        _argmin_body,
        grid=(NBLK,),
        in_specs=[
            pl.BlockSpec((BLK, DIM), lambda i: (i, 0)),
            pl.BlockSpec((DIM, NPROTO), lambda i: (0, 0)),
            pl.BlockSpec((BLK, 1), lambda i: (i, 0)),
            pl.BlockSpec((1, NPROTO), lambda i: (0, 0)),
        ],
        out_specs=[
            pl.BlockSpec((BLK, 1), lambda i: (i, 0)),
            pl.BlockSpec((1, 1), lambda i: (0, 0)),
        ],
        out_shape=[
            jax.ShapeDtypeStruct((N_TOKENS, 1), jnp.int32),
            jax.ShapeDtypeStruct((1, 1), jnp.float32),
        ],
    )(x, prototypes, sx, sp)


def _sc_gather_body(table_hbm, idx_hbm, out_hbm, idx_v, rows_v, sem):
    wid = lax.axis_index("s") * 2 + lax.axis_index("c")
    base = wid * BPW
    pltpu.sync_copy(idx_hbm.at[wid], idx_v)
    copies = [
        pltpu.async_copy(table_hbm.at[idx_v.at[j]],
                         rows_v.at[pl.ds(j * IDX_CHUNK, IDX_CHUNK)], sem)
        for j in range(NCHUNK)
    ]
    for c in copies:
        c.wait()
    pltpu.sync_copy(rows_v, out_hbm.at[pl.ds(base, BPW)])


def _gather_call(prototypes, idx):
    mesh = plsc.VectorSubcoreMesh(core_axis_name="c", subcore_axis_name="s")
    f = pl.kernel(
        _sc_gather_body,
        out_type=jax.ShapeDtypeStruct((N_TOKENS, DIM), jnp.float32),
        mesh=mesh,
        scratch_types=[
            pltpu.VMEM((NCHUNK, IDX_CHUNK), jnp.int32),
            pltpu.VMEM((BPW, DIM), jnp.float32),
            pltpu.SemaphoreType.DMA,
        ],
        compiler_params=pltpu.CompilerParams(use_tc_tiling_on_sc=False),
    )
    return f(prototypes, idx)


def kernel(x, prototypes):
    sx = jnp.sum(x ** 2, axis=1, keepdims=True)
    sp = jnp.sum(prototypes ** 2, axis=1)
    idx2, loss = _argmin_call(x, (prototypes + prototypes).T, sx,
                              sp.reshape(1, NPROTO))
    idx = idx2.reshape(NUM_WORKERS, NCHUNK, IDX_CHUNK)
    quantized_st = _gather_call(prototypes, idx)
    return (quantized_st, loss[0, 0], prototypes)
